# baseline (speedup 1.0000x reference)
import functools

import jax
import jax.numpy as jnp
from jax import lax
from jax.experimental import pallas as pl
from jax.experimental.pallas import tpu as pltpu
from jax.experimental.pallas import tpu_sc as plsc

N_NODES = 10000
D_FEAT = 128
N_EDGES = 320000
BN_EPS = 1e-5

_NC = 2
_NS = 16
_SEG = 2048
_NSEG = 80
_EPAD = _NC * _NSEG * _SEG
_NPAD = 10240
_RT = _NPAD // _NS
_K = 128
_MBUF = _SEG + _K


def _make_sc_agg():
    mesh = plsc.VectorSubcoreMesh(core_axis_name="c", subcore_axis_name="s")

    @functools.partial(
        pl.kernel,
        mesh=mesh,
        out_type=jax.ShapeDtypeStruct((_NC, _NPAD, D_FEAT), jnp.float32),
        compiler_params=pltpu.CompilerParams(needs_layout_passes=False),
        scratch_types=[
            pltpu.VMEM((_SEG,), jnp.int32),
            pltpu.VMEM((_SEG,), jnp.int32),
            pltpu.VMEM((_MBUF,), jnp.int32),
            pltpu.VMEM((_MBUF,), jnp.int32),
            pltpu.VMEM((_K,), jnp.int32),
            pltpu.VMEM((_K,), jnp.int32),
            pltpu.VMEM((_K, D_FEAT), jnp.float32),
            pltpu.VMEM((_K, D_FEAT), jnp.float32),
            pltpu.VMEM((_RT, D_FEAT), jnp.float32),
            pltpu.SemaphoreType.DMA,
            pltpu.SemaphoreType.DMA,
            pltpu.SemaphoreType.DMA,
            pltpu.SemaphoreType.DMA,
        ],
    )
    def sc_agg(x_hbm, combo_hbm, out_hbm,
               sega, segb, srcbuf, dstbuf, scha, schb, gba, gbb, acc,
               sema, semb, semga, semgb):
        cid = lax.axis_index("c")
        sid = lax.axis_index("s")
        lo = sid * _RT

        z16 = jnp.zeros((16,), jnp.float32)

        def zbody(i, _):
            r = jnp.int32(i) // (D_FEAT // 16)
            c = jnp.int32(i) % (D_FEAT // 16)
            acc[r, pl.ds(c * 16, 16)] = z16
            return 0

        lax.fori_loop(jnp.int32(0), jnp.int32(_RT * D_FEAT // 16),
                      zbody, 0)

        zi16 = jnp.zeros((16,), jnp.int32)

        def zidx(i, _):
            srcbuf[pl.ds(jnp.int32(i) * 16, 16)] = zi16
            dstbuf[pl.ds(jnp.int32(i) * 16, 16)] = zi16
            return 0

        lax.fori_loop(jnp.int32(0), jnp.int32(_MBUF // 16), zidx, 0)

        def scan_seg(seg, i, ptr):
            cv = seg[pl.ds(i * 16, 16)]
            dstv = lax.bitwise_and(cv, jnp.int32(0xFFFF))
            srcv = lax.shift_right_logical(cv, jnp.int32(16))
            m = jnp.logical_and(dstv >= lo, dstv < lo + _RT)
            plsc.store_compressed(srcbuf.at[pl.ds(ptr, 16)], srcv, mask=m)
            plsc.store_compressed(dstbuf.at[pl.ds(ptr, 16)], dstv - lo, mask=m)
            cnt = plsc.all_reduce_population_count(m)[0]
            return ptr + cnt

        def process_seg(seg):
            mcnt = lax.fori_loop(
                jnp.int32(0), jnp.int32(_SEG // 16),
                lambda i, p: scan_seg(seg, jnp.int32(i), p), jnp.int32(0))
            z16 = jnp.zeros((16,), jnp.int32)
            for v in range(_K // 16):
                srcbuf[pl.ds(mcnt + v * 16, 16)] = jnp.full(
                    (16,), N_NODES, jnp.int32)
                dstbuf[pl.ds(mcnt + v * 16, 16)] = z16

            def start_gather(c, sch, gb, sem):
                base = c * _K
                for v in range(_K // 16):
                    sch[pl.ds(v * 16, 16)] = srcbuf[pl.ds(base + v * 16, 16)]
                pltpu.async_copy(x_hbm.at[sch], gb, sem)

            def add_chunk(c, gb):
                base = c * _K

                def group_body(g, _):
                    g32 = jnp.int32(g)
                    dv = dstbuf[pl.ds(base + g32 * 16, 16)]
                    for l in range(16):
                        dstl = dv[l]
                        e = g32 * 16 + l
                        for v in range(D_FEAT // 16):
                            sl = pl.ds(v * 16, 16)
                            acc[dstl, sl] = acc[dstl, sl] + gb[e, sl]
                    return 0

                lax.fori_loop(jnp.int32(0), jnp.int32(_K // 16), group_body, 0)

            nq = jnp.int32(2)

            @pl.when(nq > 0)
            def _():
                start_gather(jnp.int32(0), scha, gba, semga)

            def cpair(p, _):
                c0 = jnp.int32(p) * 2
                c1 = c0 + 1

                @pl.when(c1 < nq)
                def _():
                    start_gather(c1, schb, gbb, semgb)

                pltpu.make_async_copy(x_hbm.at[scha], gba, semga).wait()
                add_chunk(c0, gba)

                @pl.when(c1 + 1 < nq)
                def _():
                    start_gather(c1 + 1, scha, gba, semga)

                @pl.when(c1 < nq)
                def _():
                    pltpu.make_async_copy(x_hbm.at[schb], gbb, semgb).wait()
                    add_chunk(c1, gbb)

                return 0

            lax.fori_loop(jnp.int32(0), (nq + 1) // 2, cpair, 0)

        pltpu.async_copy(combo_hbm.at[cid, jnp.int32(0)], sega, sema)

        def pair_body(p, _):
            s = jnp.int32(p) * 2
            pltpu.async_copy(combo_hbm.at[cid, s + 1], segb, semb)
            pltpu.make_async_copy(combo_hbm.at[cid, s], sega, sema).wait()
            process_seg(sega)

            @pl.when(s + 2 < _NSEG)
            def _():
                pltpu.async_copy(combo_hbm.at[cid, s + 2], sega, sema)

            pltpu.make_async_copy(combo_hbm.at[cid, s + 1], segb, semb).wait()
            process_seg(segb)
            return 0

        lax.fori_loop(jnp.int32(0), jnp.int32(_NSEG // 2), pair_body, 0)

        pltpu.sync_copy(acc.at[pl.ds(0, _RT)],
                        out_hbm.at[cid, pl.ds(lo, _RT)])

    return sc_agg


def _tc_finish(x_ref, agg_ref, w_ref, b_ref, g_ref, bt_ref, out_ref):
    h = x_ref[...] + agg_ref[0, :N_NODES] + agg_ref[1, :N_NODES]
    lin = lax.dot_general(h, w_ref[...], (((1,), (1,)), ((), ())),
                          preferred_element_type=jnp.float32) + b_ref[...]
    mean = jnp.mean(lin, axis=0, keepdims=True)
    cent = lin - mean
    var = jnp.mean(cent * cent, axis=0, keepdims=True)
    out_ref[...] = cent * lax.rsqrt(var + BN_EPS) * g_ref[...] + bt_ref[...]


def kernel(x, edge_index, W, b, gamma, beta):
    ei = edge_index.astype(jnp.int32)
    pad = _EPAD - N_EDGES
    src = jnp.concatenate([ei[0], jnp.full((pad,), N_NODES, jnp.int32)])
    dst = jnp.concatenate(
        [ei[1], (jnp.arange(pad, dtype=jnp.int32) % _NPAD)])
    combo = jnp.bitwise_or(jnp.left_shift(src, 16), dst)
    combo3 = combo.reshape(_NC, _NSEG, _SEG)
    x_pad = jnp.concatenate([x, jnp.zeros((8, D_FEAT), jnp.float32)])

    agg = _make_sc_agg()(x_pad, combo3)

    out = pl.pallas_call(
        _tc_finish,
        out_shape=jax.ShapeDtypeStruct((N_NODES, D_FEAT), jnp.float32),
    )(x, agg, W, b.reshape(1, D_FEAT), gamma.reshape(1, D_FEAT),
      beta.reshape(1, D_FEAT))
    return out


# speedup vs baseline: 9.6754x; 9.6754x over previous
import functools

import jax
import jax.numpy as jnp
from jax import lax
from jax.experimental import pallas as pl
from jax.experimental.pallas import tpu as pltpu
from jax.experimental.pallas import tpu_sc as plsc

N_NODES = 10000
D_FEAT = 128
N_EDGES = 320000
BN_EPS = 1e-5

_NC = 2
_NS = 16
_SEG = 2048
_NSEG = 80
_EPAD = _NC * _NSEG * _SEG
_NPAD = 10240
_RT = _NPAD // _NS
_K = 128
_MBUF = _SEG + _K


def _make_sc_agg():
    mesh = plsc.VectorSubcoreMesh(core_axis_name="c", subcore_axis_name="s")

    @functools.partial(
        pl.kernel,
        mesh=mesh,
        out_type=jax.ShapeDtypeStruct((_NC, _NPAD, D_FEAT), jnp.float32),
        compiler_params=pltpu.CompilerParams(needs_layout_passes=False),
        scratch_types=[
            pltpu.VMEM((_SEG,), jnp.int32),
            pltpu.VMEM((_SEG,), jnp.int32),
            pltpu.VMEM((_MBUF,), jnp.int32),
            pltpu.VMEM((_MBUF,), jnp.int32),
            pltpu.VMEM((_K,), jnp.int32),
            pltpu.VMEM((_K,), jnp.int32),
            pltpu.VMEM((_K, D_FEAT), jnp.float32),
            pltpu.VMEM((_K, D_FEAT), jnp.float32),
            pltpu.VMEM((_RT, D_FEAT), jnp.float32),
            pltpu.SemaphoreType.DMA,
            pltpu.SemaphoreType.DMA,
            pltpu.SemaphoreType.DMA,
            pltpu.SemaphoreType.DMA,
        ],
    )
    def sc_agg(x_hbm, combo_hbm, out_hbm,
               sega, segb, srcbuf, dstbuf, scha, schb, gba, gbb, acc,
               sema, semb, semga, semgb):
        cid = lax.axis_index("c")
        sid = lax.axis_index("s")
        lo = sid * _RT

        z16 = jnp.zeros((16,), jnp.float32)

        def zbody(i, _):
            r = jnp.int32(i) // (D_FEAT // 16)
            c = jnp.int32(i) % (D_FEAT // 16)
            acc[r, pl.ds(c * 16, 16)] = z16
            return 0

        lax.fori_loop(jnp.int32(0), jnp.int32(_RT * D_FEAT // 16),
                      zbody, 0)

        zi16 = jnp.zeros((16,), jnp.int32)

        def zidx(i, _):
            i32 = jnp.int32(i) * 16
            srcbuf[pl.ds(i32, 16)] = lax.iota(jnp.int32, 16) + lax.rem(
                i32, jnp.int32(4096))
            dstbuf[pl.ds(i32, 16)] = zi16
            return 0

        lax.fori_loop(jnp.int32(0), jnp.int32(_MBUF // 16), zidx, 0)

        def scan_seg(seg, i, ptr):
            cv = seg[pl.ds(i * 16, 16)]
            dstv = lax.bitwise_and(cv, jnp.int32(0xFFFF))
            srcv = lax.shift_right_logical(cv, jnp.int32(16))
            m = jnp.logical_and(dstv >= lo, dstv < lo + _RT)
            plsc.store_compressed(srcbuf.at[pl.ds(ptr, 16)], srcv, mask=m)
            plsc.store_compressed(dstbuf.at[pl.ds(ptr, 16)], dstv - lo, mask=m)
            cnt = plsc.all_reduce_population_count(m)[0]
            return ptr + cnt

        def process_seg(seg):
            mcnt = lax.fori_loop(
                jnp.int32(0), jnp.int32(_SEG // 16),
                lambda i, p: scan_seg(seg, jnp.int32(i), p), jnp.int32(0))
            z16 = jnp.zeros((16,), jnp.int32)
            iota16 = lax.iota(jnp.int32, 16)
            for v in range(_K // 16):
                srcbuf[pl.ds(mcnt + v * 16, 16)] = iota16 + (v * 16)
                dstbuf[pl.ds(mcnt + v * 16, 16)] = z16

            def start_gather(c, sch, gb, sem):
                base = c * _K
                for v in range(_K // 16):
                    sch[pl.ds(v * 16, 16)] = srcbuf[pl.ds(base + v * 16, 16)]
                pltpu.async_copy(x_hbm.at[sch], gb, sem)

            def add_chunk(c, gb):
                base = c * _K

                def group_body(g, _):
                    g32 = jnp.int32(g)
                    dv = dstbuf[pl.ds(base + g32 * 16, 16)]
                    for l in range(16):
                        dstl = dv[l]
                        e = g32 * 16 + l
                        for v in range(D_FEAT // 16):
                            sl = pl.ds(v * 16, 16)
                            acc[dstl, sl] = acc[dstl, sl] + gb[e, sl]
                    return 0

                lax.fori_loop(jnp.int32(0), jnp.int32(_K // 16), group_body, 0)

            nq = jnp.int32(2)

            @pl.when(nq > 0)
            def _():
                start_gather(jnp.int32(0), scha, gba, semga)

            def cpair(p, _):
                c0 = jnp.int32(p) * 2
                c1 = c0 + 1

                @pl.when(c1 < nq)
                def _():
                    start_gather(c1, schb, gbb, semgb)

                pltpu.make_async_copy(x_hbm.at[scha], gba, semga).wait()
                add_chunk(c0, gba)

                @pl.when(c1 + 1 < nq)
                def _():
                    start_gather(c1 + 1, scha, gba, semga)

                @pl.when(c1 < nq)
                def _():
                    pltpu.make_async_copy(x_hbm.at[schb], gbb, semgb).wait()
                    add_chunk(c1, gbb)

                return 0

            lax.fori_loop(jnp.int32(0), (nq + 1) // 2, cpair, 0)

        pltpu.async_copy(combo_hbm.at[cid, jnp.int32(0)], sega, sema)

        def pair_body(p, _):
            s = jnp.int32(p) * 2
            pltpu.async_copy(combo_hbm.at[cid, s + 1], segb, semb)
            pltpu.make_async_copy(combo_hbm.at[cid, s], sega, sema).wait()
            process_seg(sega)

            @pl.when(s + 2 < _NSEG)
            def _():
                pltpu.async_copy(combo_hbm.at[cid, s + 2], sega, sema)

            pltpu.make_async_copy(combo_hbm.at[cid, s + 1], segb, semb).wait()
            process_seg(segb)
            return 0

        lax.fori_loop(jnp.int32(0), jnp.int32(_NSEG // 2), pair_body, 0)

        pltpu.sync_copy(acc.at[pl.ds(0, _RT)],
                        out_hbm.at[cid, pl.ds(lo, _RT)])

    return sc_agg


def _tc_finish(x_ref, agg_ref, w_ref, b_ref, g_ref, bt_ref, out_ref):
    h = x_ref[...] + agg_ref[0, :N_NODES] + agg_ref[1, :N_NODES]
    lin = lax.dot_general(h, w_ref[...], (((1,), (1,)), ((), ())),
                          preferred_element_type=jnp.float32) + b_ref[...]
    mean = jnp.mean(lin, axis=0, keepdims=True)
    cent = lin - mean
    var = jnp.mean(cent * cent, axis=0, keepdims=True)
    out_ref[...] = cent * lax.rsqrt(var + BN_EPS) * g_ref[...] + bt_ref[...]


def kernel(x, edge_index, W, b, gamma, beta):
    ei = edge_index.astype(jnp.int32)
    pad = _EPAD - N_EDGES
    src = jnp.concatenate([ei[0], jnp.full((pad,), N_NODES, jnp.int32)])
    dst = jnp.concatenate(
        [ei[1], (jnp.arange(pad, dtype=jnp.int32) % _NPAD)])
    combo = jnp.bitwise_or(jnp.left_shift(src, 16), dst)
    combo3 = combo.reshape(_NC, _NSEG, _SEG)
    x_pad = jnp.concatenate([x, jnp.zeros((8, D_FEAT), jnp.float32)])

    agg = _make_sc_agg()(x_pad, combo3)

    out = pl.pallas_call(
        _tc_finish,
        out_shape=jax.ShapeDtypeStruct((N_NODES, D_FEAT), jnp.float32),
    )(x, agg, W, b.reshape(1, D_FEAT), gamma.reshape(1, D_FEAT),
      beta.reshape(1, D_FEAT))
    return out


# speedup vs baseline: 84.6376x; 8.7477x over previous
import functools

import jax
import jax.numpy as jnp
from jax import lax
from jax.experimental import pallas as pl
from jax.experimental.pallas import tpu as pltpu
from jax.experimental.pallas import tpu_sc as plsc

N_NODES = 10000
D_FEAT = 128
N_EDGES = 320000
BN_EPS = 1e-5

_NC = 2
_NS = 16
_NW = _NC * _NS
_K = 128
_CH = 80
_EW = _K * _CH
_EPAD = _EW * _NW
_NPAD = 10240
_RT = _NPAD // _NS


def _make_sc_agg():
    mesh = plsc.VectorSubcoreMesh(core_axis_name="c", subcore_axis_name="s")

    @functools.partial(
        pl.kernel,
        mesh=mesh,
        out_type=jax.ShapeDtypeStruct((_NC, _NPAD, D_FEAT), jnp.float32),
        scratch_types=[
            pltpu.VMEM((_CH, _K), jnp.int32),
            pltpu.VMEM((_K,), jnp.int32),
            pltpu.VMEM((_K,), jnp.int32),
            pltpu.VMEM((_K,), jnp.int32),
            pltpu.VMEM((_K,), jnp.int32),
            pltpu.VMEM((_K, D_FEAT), jnp.float32),
            pltpu.VMEM((_K, D_FEAT), jnp.float32),
            pltpu.VMEM_SHARED((_NPAD, D_FEAT), jnp.float32),
            pltpu.SemaphoreType.DMA,
            pltpu.SemaphoreType.DMA,
        ],
    )
    def sc_agg(x_hbm, combo_hbm, out_hbm,
               combo_v, src0, src1, dst0, dst1, buf0, buf1, agg, sem0, sem1):
        cid = lax.axis_index("c")
        sid = lax.axis_index("s")
        wid = sid * _NC + cid

        z16 = jnp.zeros((16,), jnp.float32)

        def zbody(i, _):
            r = jnp.int32(i) // (D_FEAT // 16)
            c = jnp.int32(i) % (D_FEAT // 16)
            buf0[r, pl.ds(c * 16, 16)] = z16
            return 0

        lax.fori_loop(jnp.int32(0), jnp.int32(_K * D_FEAT // 16), zbody, 0)
        for q in range(_RT // _K):
            pltpu.sync_copy(buf0, agg.at[pl.ds(sid * _RT + q * _K, _K)])

        pltpu.sync_copy(combo_hbm.at[wid], combo_v)
        plsc.subcore_barrier()

        def unpack(j, src_c, dst_c):
            for v in range(_K // 16):
                cv = combo_v[j, pl.ds(v * 16, 16)]
                src_c[pl.ds(v * 16, 16)] = lax.shift_right_logical(
                    cv, jnp.int32(16))
                dst_c[pl.ds(v * 16, 16)] = lax.bitwise_and(cv, jnp.int32(0xFFFF))

        unpack(jnp.int32(0), src0, dst0)
        pltpu.async_copy(x_hbm.at[src0], buf0, sem0)

        def body(jj, _):
            j = jnp.int32(jj) * 2
            unpack(j + 1, src1, dst1)
            pltpu.async_copy(x_hbm.at[src1], buf1, sem1)
            pltpu.make_async_copy(x_hbm.at[src0], buf0, sem0).wait()
            pltpu.sync_copy(buf0, agg.at[dst0], add=True)

            @pl.when(j + 2 < _CH)
            def _():
                unpack(j + 2, src0, dst0)
                pltpu.async_copy(x_hbm.at[src0], buf0, sem0)

            pltpu.make_async_copy(x_hbm.at[src1], buf1, sem1).wait()
            pltpu.sync_copy(buf1, agg.at[dst1], add=True)
            return 0

        lax.fori_loop(jnp.int32(0), jnp.int32(_CH // 2), body, 0)
        plsc.subcore_barrier()

        pltpu.sync_copy(agg.at[pl.ds(sid * _RT, _RT)],
                        out_hbm.at[cid, pl.ds(sid * _RT, _RT)])

    return sc_agg


def _tc_finish(x_ref, agg_ref, w_ref, b_ref, g_ref, bt_ref, out_ref):
    h = x_ref[...] + agg_ref[0, :N_NODES] + agg_ref[1, :N_NODES]
    lin = lax.dot_general(h, w_ref[...], (((1,), (1,)), ((), ())),
                          preferred_element_type=jnp.float32) + b_ref[...]
    mean = jnp.mean(lin, axis=0, keepdims=True)
    cent = lin - mean
    var = jnp.mean(cent * cent, axis=0, keepdims=True)
    out_ref[...] = cent * lax.rsqrt(var + BN_EPS) * g_ref[...] + bt_ref[...]


def kernel(x, edge_index, W, b, gamma, beta):
    ei = edge_index.astype(jnp.int32)
    pad = _EPAD - N_EDGES
    pad_i = jnp.arange(pad, dtype=jnp.int32)
    src = jnp.concatenate([ei[0], pad_i % N_NODES])
    dst = jnp.concatenate([ei[1], N_NODES + pad_i % (_NPAD - N_NODES)])
    combo = jnp.bitwise_or(jnp.left_shift(src, 16), dst)
    combo3 = combo.reshape(_NW, _CH, _K)

    agg = _make_sc_agg()(x, combo3)

    out = pl.pallas_call(
        _tc_finish,
        out_shape=jax.ShapeDtypeStruct((N_NODES, D_FEAT), jnp.float32),
    )(x, agg, W, b.reshape(1, D_FEAT), gamma.reshape(1, D_FEAT),
      beta.reshape(1, D_FEAT))
    return out
